# Initial kernel scaffold; baseline (speedup 1.0000x reference)
import jax
import jax.numpy as jnp
from jax.experimental import pallas as pl


def kernel(x, edge_index, edge_attr, params):
    raise NotImplementedError("write your pallas kernel here")


# speedup vs baseline: 3.3901x; 3.3901x over previous
import functools

import jax
import jax.numpy as jnp
from jax import lax
from jax.experimental import pallas as pl
from jax.experimental.pallas import tpu as pltpu
from jax.experimental.pallas import tpu_sc as plsc

N = 10000
E = 320000
D = 128
EPS = 1e-5

NC = 2
NS = 16
NW = NC * NS
PER_TILE = E // NW
C = 80
CHUNKS = PER_TILE // C
N_PAD = 10240
ROWS_PT = N_PAD // NS
CW = 128
VEC = 16


_MESH = plsc.VectorSubcoreMesh(core_axis_name="c", subcore_axis_name="s")


def _sc_pass_body(t_hbm, src_hbm, dst_hbm, x_hbm, s_out,
                  src_v, dst_v, t_v, g_v, s_sh, sem):
    c = lax.axis_index("c")
    s = lax.axis_index("s")
    wid = c * NS + s
    ebase = wid * PER_TILE
    rows0 = s * ROWS_PT

    pltpu.sync_copy(dst_hbm.at[wid], dst_v)

    def _zfill(i, _):
        for j in range(D // VEC):
            g_v[i, pl.ds(j * VEC, VEC)] = jnp.zeros((VEC,), jnp.float32)
        return 0
    lax.fori_loop(0, C, _zfill, 0)
    for k in range(ROWS_PT // C):
        pltpu.sync_copy(g_v, s_sh.at[pl.ds(rows0 + k * C, C)])
    plsc.subcore_barrier()

    def _chunk(i, _):
        eoff = ebase + i * C
        pltpu.sync_copy(src_hbm.at[pl.ds(eoff, C)], src_v)
        pltpu.sync_copy(t_hbm.at[pl.ds(eoff, C)], t_v)
        pltpu.async_copy(x_hbm.at[src_v], g_v, sem).wait()

        def _mulrow(r, _):
            for j in range(D // VEC):
                sl = pl.ds(j * VEC, VEC)
                t_v[r, sl] = t_v[r, sl] * g_v[r, sl]
            return 0
        lax.fori_loop(0, C, _mulrow, 0)

        pltpu.sync_copy(t_v, s_sh.at[dst_v.at[i]], add=True)
        return 0
    lax.fori_loop(0, CHUNKS, _chunk, 0)
    plsc.subcore_barrier()

    orow = c * N_PAD + rows0
    for k in range(ROWS_PT // C):
        pltpu.sync_copy(s_sh.at[pl.ds(rows0 + k * C, C)], t_v)
        pltpu.sync_copy(t_v, s_out.at[pl.ds(orow + k * C, C)])


_sc_pass = pl.kernel(
    _sc_pass_body,
    out_type=jax.ShapeDtypeStruct((NC * N_PAD, D), jnp.float32),
    mesh=_MESH,
    scratch_types=[
        pltpu.VMEM((C,), jnp.int32),
        pltpu.VMEM((CHUNKS, C), jnp.int32),
        pltpu.VMEM((C, D), jnp.float32),
        pltpu.VMEM((C, D), jnp.float32),
        pltpu.VMEM_SHARED((N_PAD, D), jnp.float32),
        pltpu.SemaphoreType.DMA,
    ],
)


def _sc_cnt_body(dst_hbm, cnt_out, dst_v, ones_v, cnt_sh):
    c = lax.axis_index("c")
    s = lax.axis_index("s")
    wid = c * NS + s
    rows0 = s * ROWS_PT
    pltpu.sync_copy(dst_hbm.at[wid], dst_v)

    def _cfill(val, i, _):
        for j in range(CW // VEC):
            ones_v[i, pl.ds(j * VEC, VEC)] = jnp.full((VEC,), val, jnp.float32)
        return 0
    lax.fori_loop(0, C, functools.partial(_cfill, 0.0), 0)
    for k in range(ROWS_PT // C):
        pltpu.sync_copy(ones_v, cnt_sh.at[pl.ds(rows0 + k * C, C)])
    lax.fori_loop(0, C, functools.partial(_cfill, 1.0), 0)
    plsc.subcore_barrier()

    def _chunk(i, _):
        pltpu.sync_copy(ones_v, cnt_sh.at[dst_v.at[i]], add=True)
        return 0
    lax.fori_loop(0, CHUNKS, _chunk, 0)
    plsc.subcore_barrier()

    orow = c * N_PAD + rows0
    for k in range(ROWS_PT // C):
        pltpu.sync_copy(cnt_sh.at[pl.ds(rows0 + k * C, C)], ones_v)
        pltpu.sync_copy(ones_v, cnt_out.at[pl.ds(orow + k * C, C)])


_sc_cnt = pl.kernel(
    _sc_cnt_body,
    out_type=jax.ShapeDtypeStruct((NC * N_PAD, CW), jnp.float32),
    mesh=_MESH,
    scratch_types=[
        pltpu.VMEM((CHUNKS, C), jnp.int32),
        pltpu.VMEM((C, CW), jnp.float32),
        pltpu.VMEM_SHARED((N_PAD, CW), jnp.float32),
    ],
)


BT = 2560


def _et_body(e_ref, r1, r2, r3, c1, c2, c3, t1, t2, eo):
    e = e_ref[...]
    t1[...] = jnp.dot(e, r1[...], preferred_element_type=jnp.float32) + c1[...]
    t2[...] = jnp.dot(e, r2[...], preferred_element_type=jnp.float32) + c2[...]
    eo[...] = jnp.dot(e, r3[...], preferred_element_type=jnp.float32) + c3[...]


def _edge_transform(e0, R1, R2, R3, c1, c2, c3):
    wspec = pl.BlockSpec((D, D), lambda i: (0, 0))
    bspec = pl.BlockSpec((1, D), lambda i: (0, 0))
    espec = pl.BlockSpec((BT, D), lambda i: (i, 0))
    return pl.pallas_call(
        _et_body,
        grid=(E // BT,),
        in_specs=[espec, wspec, wspec, wspec, bspec, bspec, bspec],
        out_specs=[espec, espec, espec],
        out_shape=[jax.ShapeDtypeStruct((E, D), jnp.float32)] * 3,
    )(e0, R1, R2, R3, c1.reshape(1, D), c2.reshape(1, D), c3.reshape(1, D))


def _nu_body(do_bn, s_ref, cnt_ref, x_ref, wi, bi, ws, bs, g_ref, b_ref, o_ref):
    sa = s_ref[...]
    S = sa[0:N] + sa[N_PAD:N_PAD + N]
    ca = cnt_ref[...]
    cnt = ca[0:N, 0:1] + ca[N_PAD:N_PAD + N, 0:1]
    meanh = S / jnp.maximum(cnt, 1.0)
    out = (jnp.dot(meanh, wi[...], preferred_element_type=jnp.float32)
           + jnp.minimum(cnt, 1.0) * bi[...]
           + jnp.dot(x_ref[...], ws[...], preferred_element_type=jnp.float32)
           + bs[...])
    if do_bn:
        mu = jnp.mean(out, axis=0, keepdims=True)
        var = jnp.mean((out - mu) ** 2, axis=0, keepdims=True)
        out = (out - mu) / jnp.sqrt(var + EPS) * g_ref[...] + b_ref[...]
        out = jnp.maximum(out, 0.0)
    o_ref[...] = out


def _node_update(Scat, cntcat, x, WinT, b_in, WselfT, b_self, bn_g, bn_b, do_bn):
    return pl.pallas_call(
        functools.partial(_nu_body, do_bn),
        out_shape=jax.ShapeDtypeStruct((N, D), jnp.float32),
    )(Scat, cntcat, x, WinT, b_in.reshape(1, D), WselfT,
      b_self.reshape(1, D), bn_g.reshape(1, D), bn_b.reshape(1, D))


def kernel(x, edge_index, edge_attr, params):
    p = params
    src = edge_index[0]
    dst = edge_index[1]

    W0T = p["W_rel_0"].T
    W1T = p["W_rel_1"].T
    W2T = p["W_rel_2"].T
    R1 = W0T
    c1 = p["b_rel_0"]
    R2 = R1 @ W1T
    c2 = c1 @ W1T + p["b_rel_1"]
    R3 = R2 @ W2T
    c3 = c2 @ W2T + p["b_rel_2"]

    t1, t2, e_out = _edge_transform(edge_attr, R1, R2, R3, c1, c2, c3)

    dst3 = dst.reshape(NW, CHUNKS, C)
    cnt = _sc_cnt(dst3)
    S0 = _sc_pass(edge_attr, src, dst3, x)
    x1 = _node_update(S0, cnt, x, p["W_in_0"].T, p["b_in_0"],
                      p["W_self_0"].T, p["b_self_0"],
                      p["bn_g_0"], p["bn_b_0"], True)
    S1 = _sc_pass(t1, src, dst3, x1)
    x2 = _node_update(S1, cnt, x1, p["W_in_1"].T, p["b_in_1"],
                      p["W_self_1"].T, p["b_self_1"],
                      p["bn_g_1"], p["bn_b_1"], True)
    S2 = _sc_pass(t2, src, dst3, x2)
    x3 = _node_update(S2, cnt, x2, p["W_in_2"].T, p["b_in_2"],
                      p["W_self_2"].T, p["b_self_2"],
                      p["bn_g_1"], p["bn_b_1"], False)
    return (x3, e_out)


# speedup vs baseline: 4.3661x; 1.2879x over previous
import functools

import jax
import jax.numpy as jnp
from jax import lax
from jax.experimental import pallas as pl
from jax.experimental.pallas import tpu as pltpu
from jax.experimental.pallas import tpu_sc as plsc

N = 10000
E = 320000
D = 128
EPS = 1e-5

NC = 2
NS = 16
NW = NC * NS
PER_TILE = E // NW
C = 40
CHUNKS = PER_TILE // C
N_PAD = 10240
ROWS_PT = N_PAD // NS
CW = 128
VEC = 16


_MESH = plsc.VectorSubcoreMesh(core_axis_name="c", subcore_axis_name="s")


def _sc_pass_body(t_hbm, src_hbm, dst_hbm, x_hbm, s_out,
                  src_v, dst_v, t_v0, t_v1, g_v0, g_v1, h_v0, h_v1, s_sh,
                  sem_sd0, sem_sd1, sem_g0, sem_g1, sem_w0, sem_w1):
    c = lax.axis_index("c")
    s = lax.axis_index("s")
    wid = c * NS + s
    ebase = wid * PER_TILE
    rows0 = s * ROWS_PT
    t_v = (t_v0, t_v1)
    g_v = (g_v0, g_v1)
    h_v = (h_v0, h_v1)
    sem_sd = (sem_sd0, sem_sd1)
    sem_g = (sem_g0, sem_g1)
    sem_w = (sem_w0, sem_w1)

    def _issue_loads(i, ph, b):
        pltpu.async_copy(src_hbm.at[wid, i], src_v.at[ph], sem_sd[b])
        pltpu.async_copy(dst_hbm.at[wid, i], dst_v.at[ph], sem_sd[b])
        pltpu.async_copy(t_hbm.at[pl.ds(ebase + i * C, C)], t_v[b], sem_sd[b])

    def _wait_loads(i, ph, b):
        pltpu.make_async_copy(src_hbm.at[wid, i], src_v.at[ph], sem_sd[b]).wait()
        pltpu.make_async_copy(dst_hbm.at[wid, i], dst_v.at[ph], sem_sd[b]).wait()
        pltpu.make_async_copy(t_hbm.at[pl.ds(ebase + i * C, C)], t_v[b],
                              sem_sd[b]).wait()

    def _issue_gather(ph, b):
        pltpu.async_copy(x_hbm.at[src_v.at[ph]], g_v[b], sem_g[b])

    def _wait_gather(ph, b):
        pltpu.make_async_copy(x_hbm.at[src_v.at[ph]], g_v[b], sem_g[b]).wait()

    def _issue_scatter(ph, b):
        pltpu.async_copy(h_v[b], s_sh.at[dst_v.at[ph]], sem_w[b], add=True)

    def _wait_scatter(ph, b):
        pltpu.make_async_copy(h_v[b], s_sh.at[dst_v.at[ph]], sem_w[b]).wait()

    def _mul(b):
        tb, gb, hb = t_v[b], g_v[b], h_v[b]

        def _mulrow(r, _):
            for j in range(D // VEC):
                sl = pl.ds(j * VEC, VEC)
                hb[r, sl] = tb[r, sl] * gb[r, sl]
            return 0
        lax.fori_loop(0, C, _mulrow, 0)

    def _zfill(i, _):
        for j in range(D // VEC):
            h_v0[i, pl.ds(j * VEC, VEC)] = jnp.zeros((VEC,), jnp.float32)
        return 0
    lax.fori_loop(0, C, _zfill, 0)
    for k in range(ROWS_PT // C):
        pltpu.sync_copy(h_v0, s_sh.at[pl.ds(rows0 + k * C, C)])
    plsc.subcore_barrier()

    _issue_loads(0, 0, 0)
    _issue_loads(1, 1, 1)
    _wait_loads(0, 0, 0)
    _issue_gather(0, 0)

    def _quad(qi, _):
        for u in range(4):
            b = u % 2
            i = qi * 4 + u
            _wait_gather(u, b)
            if u >= 2:
                _wait_scatter(u, b)
            else:
                @pl.when(qi >= 1)
                def _():
                    _wait_scatter(u, b)
            _mul(b)
            _issue_scatter(u, b)
            _issue_loads(i + 2, (u + 2) % 4, b)
            _wait_loads(i + 1, (u + 1) % 4, 1 - b)
            _issue_gather((u + 1) % 4, 1 - b)
        return 0
    lax.fori_loop(0, CHUNKS // 4, _quad, 0)

    _wait_gather(0, 0)
    _wait_scatter(0, 0)
    _mul(0)
    _issue_scatter(0, 0)
    _wait_loads(CHUNKS - 1, 1, 1)
    _issue_gather(1, 1)
    _wait_gather(1, 1)
    _wait_scatter(1, 1)
    _mul(1)
    _issue_scatter(1, 1)
    _wait_scatter(0, 0)
    _wait_scatter(1, 1)
    plsc.subcore_barrier()

    orow = c * N_PAD + rows0
    for k in range(ROWS_PT // C):
        pltpu.sync_copy(s_sh.at[pl.ds(rows0 + k * C, C)], t_v0)
        pltpu.sync_copy(t_v0, s_out.at[pl.ds(orow + k * C, C)])


_sc_pass = pl.kernel(
    _sc_pass_body,
    out_type=jax.ShapeDtypeStruct((NC * N_PAD, D), jnp.float32),
    mesh=_MESH,
    scratch_types=[
        pltpu.VMEM((4, C), jnp.int32),
        pltpu.VMEM((4, C), jnp.int32),
        pltpu.VMEM((C, D), jnp.float32),
        pltpu.VMEM((C, D), jnp.float32),
        pltpu.VMEM((C, D), jnp.float32),
        pltpu.VMEM((C, D), jnp.float32),
        pltpu.VMEM((C, D), jnp.float32),
        pltpu.VMEM((C, D), jnp.float32),
        pltpu.VMEM_SHARED((N_PAD, D), jnp.float32),
        pltpu.SemaphoreType.DMA,
        pltpu.SemaphoreType.DMA,
        pltpu.SemaphoreType.DMA,
        pltpu.SemaphoreType.DMA,
        pltpu.SemaphoreType.DMA,
        pltpu.SemaphoreType.DMA,
    ],
)


def _sc_cnt_body(dst_hbm, cnt_out, dst_v, ones_v, cnt_sh):
    c = lax.axis_index("c")
    s = lax.axis_index("s")
    wid = c * NS + s
    rows0 = s * ROWS_PT
    pltpu.sync_copy(dst_hbm.at[wid], dst_v)

    def _cfill(val, i, _):
        for j in range(CW // VEC):
            ones_v[i, pl.ds(j * VEC, VEC)] = jnp.full((VEC,), val, jnp.float32)
        return 0
    lax.fori_loop(0, C, functools.partial(_cfill, 0.0), 0)
    for k in range(ROWS_PT // C):
        pltpu.sync_copy(ones_v, cnt_sh.at[pl.ds(rows0 + k * C, C)])
    lax.fori_loop(0, C, functools.partial(_cfill, 1.0), 0)
    plsc.subcore_barrier()

    def _chunk(i, _):
        pltpu.sync_copy(ones_v, cnt_sh.at[dst_v.at[i]], add=True)
        return 0
    lax.fori_loop(0, CHUNKS, _chunk, 0)
    plsc.subcore_barrier()

    orow = c * N_PAD + rows0
    for k in range(ROWS_PT // C):
        pltpu.sync_copy(cnt_sh.at[pl.ds(rows0 + k * C, C)], ones_v)
        pltpu.sync_copy(ones_v, cnt_out.at[pl.ds(orow + k * C, C)])


_sc_cnt = pl.kernel(
    _sc_cnt_body,
    out_type=jax.ShapeDtypeStruct((NC * N_PAD, CW), jnp.float32),
    mesh=_MESH,
    scratch_types=[
        pltpu.VMEM((CHUNKS, C), jnp.int32),
        pltpu.VMEM((C, CW), jnp.float32),
        pltpu.VMEM_SHARED((N_PAD, CW), jnp.float32),
    ],
)


BT = 2560


def _et_body(e_ref, r1, r2, r3, c1, c2, c3, t1, t2, eo):
    e = e_ref[...]
    t1[...] = jnp.dot(e, r1[...], preferred_element_type=jnp.float32) + c1[...]
    t2[...] = jnp.dot(e, r2[...], preferred_element_type=jnp.float32) + c2[...]
    eo[...] = jnp.dot(e, r3[...], preferred_element_type=jnp.float32) + c3[...]


def _edge_transform(e0, R1, R2, R3, c1, c2, c3):
    wspec = pl.BlockSpec((D, D), lambda i: (0, 0))
    bspec = pl.BlockSpec((1, D), lambda i: (0, 0))
    espec = pl.BlockSpec((BT, D), lambda i: (i, 0))
    return pl.pallas_call(
        _et_body,
        grid=(E // BT,),
        in_specs=[espec, wspec, wspec, wspec, bspec, bspec, bspec],
        out_specs=[espec, espec, espec],
        out_shape=[jax.ShapeDtypeStruct((E, D), jnp.float32)] * 3,
    )(e0, R1, R2, R3, c1.reshape(1, D), c2.reshape(1, D), c3.reshape(1, D))


def _nu_body(do_bn, s_ref, cnt_ref, x_ref, wi, bi, ws, bs, g_ref, b_ref, o_ref):
    sa = s_ref[...]
    S = sa[0:N] + sa[N_PAD:N_PAD + N]
    ca = cnt_ref[...]
    cnt = ca[0:N, 0:1] + ca[N_PAD:N_PAD + N, 0:1]
    meanh = S / jnp.maximum(cnt, 1.0)
    out = (jnp.dot(meanh, wi[...], preferred_element_type=jnp.float32)
           + jnp.minimum(cnt, 1.0) * bi[...]
           + jnp.dot(x_ref[...], ws[...], preferred_element_type=jnp.float32)
           + bs[...])
    if do_bn:
        mu = jnp.mean(out, axis=0, keepdims=True)
        var = jnp.mean((out - mu) ** 2, axis=0, keepdims=True)
        out = (out - mu) / jnp.sqrt(var + EPS) * g_ref[...] + b_ref[...]
        out = jnp.maximum(out, 0.0)
    o_ref[...] = out


def _node_update(Scat, cntcat, x, WinT, b_in, WselfT, b_self, bn_g, bn_b, do_bn):
    return pl.pallas_call(
        functools.partial(_nu_body, do_bn),
        out_shape=jax.ShapeDtypeStruct((N, D), jnp.float32),
    )(Scat, cntcat, x, WinT, b_in.reshape(1, D), WselfT,
      b_self.reshape(1, D), bn_g.reshape(1, D), bn_b.reshape(1, D))


def kernel(x, edge_index, edge_attr, params):
    p = params
    src = edge_index[0]
    dst = edge_index[1]

    W0T = p["W_rel_0"].T
    W1T = p["W_rel_1"].T
    W2T = p["W_rel_2"].T
    R1 = W0T
    c1 = p["b_rel_0"]
    R2 = R1 @ W1T
    c2 = c1 @ W1T + p["b_rel_1"]
    R3 = R2 @ W2T
    c3 = c2 @ W2T + p["b_rel_2"]

    t1, t2, e_out = _edge_transform(edge_attr, R1, R2, R3, c1, c2, c3)

    src3 = src.reshape(NW, CHUNKS, C)
    dst3 = dst.reshape(NW, CHUNKS, C)
    cnt = _sc_cnt(dst3)
    S0 = _sc_pass(edge_attr, src3, dst3, x)
    x1 = _node_update(S0, cnt, x, p["W_in_0"].T, p["b_in_0"],
                      p["W_self_0"].T, p["b_self_0"],
                      p["bn_g_0"], p["bn_b_0"], True)
    S1 = _sc_pass(t1, src3, dst3, x1)
    x2 = _node_update(S1, cnt, x1, p["W_in_1"].T, p["b_in_1"],
                      p["W_self_1"].T, p["b_self_1"],
                      p["bn_g_1"], p["bn_b_1"], True)
    S2 = _sc_pass(t2, src3, dst3, x2)
    x3 = _node_update(S2, cnt, x2, p["W_in_2"].T, p["b_in_2"],
                      p["W_self_2"].T, p["b_self_2"],
                      p["bn_g_1"], p["bn_b_1"], False)
    return (x3, e_out)


# speedup vs baseline: 5.2599x; 1.2047x over previous
import functools

import jax
import jax.numpy as jnp
from jax import lax
from jax.experimental import pallas as pl
from jax.experimental.pallas import tpu as pltpu
from jax.experimental.pallas import tpu_sc as plsc

N = 10000
E = 320000
D = 128
EPS = 1e-5

NC = 2
NS = 16
NW = NC * NS
PER_TILE = E // NW
C = 80
CHUNKS = PER_TILE // C
N_PAD = 10240
ROWS_PT = N_PAD // NS
CW = 128
VEC = 16


_MESH = plsc.VectorSubcoreMesh(core_axis_name="c", subcore_axis_name="s")


def _sc_pass_body(t_hbm, src_hbm, dst_hbm, x_hbm, s_out,
                  src_v, dst_v, t_v0, t_v1, gh_v0, gh_v1, s_sh,
                  sem_sd0, sem_sd1, sem_g0, sem_g1, sem_w0, sem_w1):
    c = lax.axis_index("c")
    s = lax.axis_index("s")
    wid = c * NS + s
    ebase = wid * PER_TILE
    rows0 = s * ROWS_PT
    t_v = (t_v0, t_v1)
    gh_v = (gh_v0, gh_v1)
    sem_sd = (sem_sd0, sem_sd1)
    sem_g = (sem_g0, sem_g1)
    sem_w = (sem_w0, sem_w1)

    def _issue_loads(i, ph, b):
        pltpu.async_copy(src_hbm.at[wid, i], src_v.at[ph], sem_sd[b])
        pltpu.async_copy(dst_hbm.at[wid, i], dst_v.at[ph], sem_sd[b])
        pltpu.async_copy(t_hbm.at[pl.ds(ebase + i * C, C)], t_v[b], sem_sd[b])

    def _wait_loads(i, ph, b):
        pltpu.make_async_copy(src_hbm.at[wid, i], src_v.at[ph], sem_sd[b]).wait()
        pltpu.make_async_copy(dst_hbm.at[wid, i], dst_v.at[ph], sem_sd[b]).wait()
        pltpu.make_async_copy(t_hbm.at[pl.ds(ebase + i * C, C)], t_v[b],
                              sem_sd[b]).wait()

    def _issue_gather(ph, b):
        pltpu.async_copy(x_hbm.at[src_v.at[ph]], gh_v[b], sem_g[b])

    def _wait_gather(ph, b):
        pltpu.make_async_copy(x_hbm.at[src_v.at[ph]], gh_v[b], sem_g[b]).wait()

    def _issue_scatter(ph, b):
        pltpu.async_copy(gh_v[b], s_sh.at[dst_v.at[ph]], sem_w[b], add=True)

    def _wait_scatter(ph, b):
        pltpu.make_async_copy(gh_v[b], s_sh.at[dst_v.at[ph]], sem_w[b]).wait()

    def _mul(b):
        tb, hb = t_v[b], gh_v[b]

        def _mulrow(r, _):
            r4 = r * 4
            for rr in range(4):
                for j in range(D // VEC):
                    sl = pl.ds(j * VEC, VEC)
                    hb[r4 + rr, sl] = tb[r4 + rr, sl] * hb[r4 + rr, sl]
            return 0
        lax.fori_loop(0, C // 4, _mulrow, 0)

    def _zfill(i, _):
        for j in range(D // VEC):
            gh_v0[i, pl.ds(j * VEC, VEC)] = jnp.zeros((VEC,), jnp.float32)
        return 0
    lax.fori_loop(0, C, _zfill, 0)
    for k in range(ROWS_PT // C):
        pltpu.sync_copy(gh_v0, s_sh.at[pl.ds(rows0 + k * C, C)])
    plsc.subcore_barrier()

    _issue_loads(0, 0, 0)
    _issue_loads(1, 1, 1)
    _wait_loads(0, 0, 0)
    _issue_gather(0, 0)

    def _quad(qi, _):
        for u in range(4):
            b = u % 2
            i = qi * 4 + u
            _wait_gather(u, b)
            _mul(b)
            _issue_scatter(u, b)

            @pl.when(i + 2 < CHUNKS)
            def _():
                _issue_loads(i + 2, (u + 2) % 4, b)
            _wait_loads(i + 1, (u + 1) % 4, 1 - b)
            if u == 0:
                @pl.when(qi >= 1)
                def _():
                    _wait_scatter((u + 3) % 4, 1 - b)
            else:
                _wait_scatter((u + 3) % 4, 1 - b)
            _issue_gather((u + 1) % 4, 1 - b)
        return 0
    lax.fori_loop(0, CHUNKS // 4, _quad, 0)

    _wait_gather(0, 0)
    _mul(0)
    _issue_scatter(0, 0)
    _wait_scatter(3, 1)
    _wait_scatter(0, 0)
    plsc.subcore_barrier()

    orow = c * N_PAD + rows0
    for k in range(ROWS_PT // C):
        pltpu.sync_copy(s_sh.at[pl.ds(rows0 + k * C, C)], t_v0)
        pltpu.sync_copy(t_v0, s_out.at[pl.ds(orow + k * C, C)])


_sc_pass = pl.kernel(
    _sc_pass_body,
    out_type=jax.ShapeDtypeStruct((NC * N_PAD, D), jnp.float32),
    mesh=_MESH,
    scratch_types=[
        pltpu.VMEM((4, C), jnp.int32),
        pltpu.VMEM((4, C), jnp.int32),
        pltpu.VMEM((C, D), jnp.float32),
        pltpu.VMEM((C, D), jnp.float32),
        pltpu.VMEM((C, D), jnp.float32),
        pltpu.VMEM((C, D), jnp.float32),
        pltpu.VMEM_SHARED((N_PAD, D), jnp.float32),
        pltpu.SemaphoreType.DMA,
        pltpu.SemaphoreType.DMA,
        pltpu.SemaphoreType.DMA,
        pltpu.SemaphoreType.DMA,
        pltpu.SemaphoreType.DMA,
        pltpu.SemaphoreType.DMA,
    ],
)


def _sc_cnt_body(dst_hbm, cnt_out, dst_v, ones_v, cnt_sh):
    c = lax.axis_index("c")
    s = lax.axis_index("s")
    wid = c * NS + s
    rows0 = s * ROWS_PT
    pltpu.sync_copy(dst_hbm.at[wid], dst_v)

    def _cfill(val, i, _):
        for j in range(CW // VEC):
            ones_v[i, pl.ds(j * VEC, VEC)] = jnp.full((VEC,), val, jnp.float32)
        return 0
    lax.fori_loop(0, C, functools.partial(_cfill, 0.0), 0)
    for k in range(ROWS_PT // C):
        pltpu.sync_copy(ones_v, cnt_sh.at[pl.ds(rows0 + k * C, C)])
    lax.fori_loop(0, C, functools.partial(_cfill, 1.0), 0)
    plsc.subcore_barrier()

    def _chunk(i, _):
        pltpu.sync_copy(ones_v, cnt_sh.at[dst_v.at[i]], add=True)
        return 0
    lax.fori_loop(0, CHUNKS, _chunk, 0)
    plsc.subcore_barrier()

    orow = c * N_PAD + rows0
    for k in range(ROWS_PT // C):
        pltpu.sync_copy(cnt_sh.at[pl.ds(rows0 + k * C, C)], ones_v)
        pltpu.sync_copy(ones_v, cnt_out.at[pl.ds(orow + k * C, C)])


_sc_cnt = pl.kernel(
    _sc_cnt_body,
    out_type=jax.ShapeDtypeStruct((NC * N_PAD, CW), jnp.float32),
    mesh=_MESH,
    scratch_types=[
        pltpu.VMEM((CHUNKS, C), jnp.int32),
        pltpu.VMEM((C, CW), jnp.float32),
        pltpu.VMEM_SHARED((N_PAD, CW), jnp.float32),
    ],
)


BT = 2560


def _et_body(e_ref, r1, r2, r3, c1, c2, c3, t1, t2, eo):
    e = e_ref[...]
    t1[...] = jnp.dot(e, r1[...], preferred_element_type=jnp.float32) + c1[...]
    t2[...] = jnp.dot(e, r2[...], preferred_element_type=jnp.float32) + c2[...]
    eo[...] = jnp.dot(e, r3[...], preferred_element_type=jnp.float32) + c3[...]


def _edge_transform(e0, R1, R2, R3, c1, c2, c3):
    wspec = pl.BlockSpec((D, D), lambda i: (0, 0))
    bspec = pl.BlockSpec((1, D), lambda i: (0, 0))
    espec = pl.BlockSpec((BT, D), lambda i: (i, 0))
    return pl.pallas_call(
        _et_body,
        grid=(E // BT,),
        in_specs=[espec, wspec, wspec, wspec, bspec, bspec, bspec],
        out_specs=[espec, espec, espec],
        out_shape=[jax.ShapeDtypeStruct((E, D), jnp.float32)] * 3,
    )(e0, R1, R2, R3, c1.reshape(1, D), c2.reshape(1, D), c3.reshape(1, D))


def _nu_body(do_bn, s_ref, cnt_ref, x_ref, wi, bi, ws, bs, g_ref, b_ref, o_ref):
    sa = s_ref[...]
    S = sa[0:N] + sa[N_PAD:N_PAD + N]
    ca = cnt_ref[...]
    cnt = ca[0:N, 0:1] + ca[N_PAD:N_PAD + N, 0:1]
    meanh = S / jnp.maximum(cnt, 1.0)
    out = (jnp.dot(meanh, wi[...], preferred_element_type=jnp.float32)
           + jnp.minimum(cnt, 1.0) * bi[...]
           + jnp.dot(x_ref[...], ws[...], preferred_element_type=jnp.float32)
           + bs[...])
    if do_bn:
        mu = jnp.mean(out, axis=0, keepdims=True)
        var = jnp.mean((out - mu) ** 2, axis=0, keepdims=True)
        out = (out - mu) / jnp.sqrt(var + EPS) * g_ref[...] + b_ref[...]
        out = jnp.maximum(out, 0.0)
    o_ref[...] = out


def _node_update(Scat, cntcat, x, WinT, b_in, WselfT, b_self, bn_g, bn_b, do_bn):
    return pl.pallas_call(
        functools.partial(_nu_body, do_bn),
        out_shape=jax.ShapeDtypeStruct((N, D), jnp.float32),
    )(Scat, cntcat, x, WinT, b_in.reshape(1, D), WselfT,
      b_self.reshape(1, D), bn_g.reshape(1, D), bn_b.reshape(1, D))


def kernel(x, edge_index, edge_attr, params):
    p = params
    src = edge_index[0]
    dst = edge_index[1]

    W0T = p["W_rel_0"].T
    W1T = p["W_rel_1"].T
    W2T = p["W_rel_2"].T
    R1 = W0T
    c1 = p["b_rel_0"]
    R2 = R1 @ W1T
    c2 = c1 @ W1T + p["b_rel_1"]
    R3 = R2 @ W2T
    c3 = c2 @ W2T + p["b_rel_2"]

    t1, t2, e_out = _edge_transform(edge_attr, R1, R2, R3, c1, c2, c3)

    src3 = src.reshape(NW, CHUNKS, C)
    dst3 = dst.reshape(NW, CHUNKS, C)
    cnt = _sc_cnt(dst3)
    S0 = _sc_pass(edge_attr, src3, dst3, x)
    x1 = _node_update(S0, cnt, x, p["W_in_0"].T, p["b_in_0"],
                      p["W_self_0"].T, p["b_self_0"],
                      p["bn_g_0"], p["bn_b_0"], True)
    S1 = _sc_pass(t1, src3, dst3, x1)
    x2 = _node_update(S1, cnt, x1, p["W_in_1"].T, p["b_in_1"],
                      p["W_self_1"].T, p["b_self_1"],
                      p["bn_g_1"], p["bn_b_1"], True)
    S2 = _sc_pass(t2, src3, dst3, x2)
    x3 = _node_update(S2, cnt, x2, p["W_in_2"].T, p["b_in_2"],
                      p["W_self_2"].T, p["b_self_2"],
                      p["bn_g_1"], p["bn_b_1"], False)
    return (x3, e_out)


# speedup vs baseline: 5.4934x; 1.0444x over previous
import functools

import jax
import jax.numpy as jnp
from jax import lax
from jax.experimental import pallas as pl
from jax.experimental.pallas import tpu as pltpu
from jax.experimental.pallas import tpu_sc as plsc

N = 10000
E = 320000
D = 128
EPS = 1e-5

NC = 2
NS = 16
NW = NC * NS
PER_TILE = E // NW
C = 80
CHUNKS = PER_TILE // C
N_PAD = 10240
ROWS_PT = N_PAD // NS
CW = 128
VEC = 16


_MESH = plsc.VectorSubcoreMesh(core_axis_name="c", subcore_axis_name="s")


def _sc_pass_body(t_bf16, t_hbm, src_hbm, dst_hbm, x_hbm, s_out,
                  src_v, dst_v, t_v0, t_v1, gh_v0, gh_v1, s_sh,
                  sem_sd0, sem_sd1, sem_g0, sem_g1, sem_w0, sem_w1):
    c = lax.axis_index("c")
    s = lax.axis_index("s")
    wid = c * NS + s
    ebase = wid * PER_TILE
    rows0 = s * ROWS_PT
    t_v = (t_v0, t_v1)
    gh_v = (gh_v0, gh_v1)
    sem_sd = (sem_sd0, sem_sd1)
    sem_g = (sem_g0, sem_g1)
    sem_w = (sem_w0, sem_w1)

    def _t_slice(i):
        return t_hbm.at[pl.ds(ebase + i * C, C)]

    def _issue_loads(i, ph, b):
        pltpu.async_copy(src_hbm.at[wid, i], src_v.at[ph], sem_sd[b])
        pltpu.async_copy(dst_hbm.at[wid, i], dst_v.at[ph], sem_sd[b])
        pltpu.async_copy(_t_slice(i), t_v[b], sem_sd[b])

    def _wait_loads(i, ph, b):
        pltpu.make_async_copy(src_hbm.at[wid, i], src_v.at[ph], sem_sd[b]).wait()
        pltpu.make_async_copy(dst_hbm.at[wid, i], dst_v.at[ph], sem_sd[b]).wait()
        pltpu.make_async_copy(_t_slice(i), t_v[b], sem_sd[b]).wait()

    def _issue_gather(ph, b):
        pltpu.async_copy(x_hbm.at[src_v.at[ph]], gh_v[b], sem_g[b])

    def _wait_gather(ph, b):
        pltpu.make_async_copy(x_hbm.at[src_v.at[ph]], gh_v[b], sem_g[b]).wait()

    def _issue_scatter(ph, b):
        pltpu.async_copy(gh_v[b], s_sh.at[dst_v.at[ph]], sem_w[b], add=True)

    def _wait_scatter(ph, b):
        pltpu.make_async_copy(gh_v[b], s_sh.at[dst_v.at[ph]], sem_w[b]).wait()

    def _mul(b):
        tb, hb = t_v[b], gh_v[b]

        def _mulrow(r, _):
            r4 = r * 4
            for rr in range(4):
                if t_bf16:
                    vsh = jnp.full((VEC,), 16, jnp.int32)
                    vmask = jnp.full((VEC,), -65536, jnp.int32)
                    for j in range(D // (2 * VEC)):
                        w = tb[r4 + rr, pl.ds(j * VEC, VEC)]
                        ta = lax.bitcast_convert_type(
                            lax.shift_left(w, vsh), jnp.float32)
                        tbb = lax.bitcast_convert_type(
                            lax.bitwise_and(w, vmask), jnp.float32)
                        sa = pl.ds(j * 2 * VEC, VEC)
                        sb = pl.ds(j * 2 * VEC + VEC, VEC)
                        hb[r4 + rr, sa] = ta * hb[r4 + rr, sa]
                        hb[r4 + rr, sb] = tbb * hb[r4 + rr, sb]
                else:
                    for j in range(D // VEC):
                        sl = pl.ds(j * VEC, VEC)
                        hb[r4 + rr, sl] = tb[r4 + rr, sl] * hb[r4 + rr, sl]
            return 0
        lax.fori_loop(0, C // 4, _mulrow, 0)

    def _zfill(i, _):
        for j in range(D // VEC):
            gh_v0[i, pl.ds(j * VEC, VEC)] = jnp.zeros((VEC,), jnp.float32)
        return 0
    lax.fori_loop(0, C, _zfill, 0)
    for k in range(ROWS_PT // C):
        pltpu.sync_copy(gh_v0, s_sh.at[pl.ds(rows0 + k * C, C)])
    plsc.subcore_barrier()

    _issue_loads(0, 0, 0)
    _issue_loads(1, 1, 1)
    _wait_loads(0, 0, 0)
    _issue_gather(0, 0)

    def _quad(qi, _):
        for u in range(4):
            b = u % 2
            i = qi * 4 + u
            _wait_gather(u, b)
            _mul(b)
            _issue_scatter(u, b)

            @pl.when(i + 2 < CHUNKS)
            def _():
                _issue_loads(i + 2, (u + 2) % 4, b)
            _wait_loads(i + 1, (u + 1) % 4, 1 - b)
            if u == 0:
                @pl.when(qi >= 1)
                def _():
                    _wait_scatter((u + 3) % 4, 1 - b)
            else:
                _wait_scatter((u + 3) % 4, 1 - b)
            _issue_gather((u + 1) % 4, 1 - b)
        return 0
    lax.fori_loop(0, CHUNKS // 4, _quad, 0)

    _wait_gather(0, 0)
    _mul(0)
    _issue_scatter(0, 0)
    _wait_scatter(3, 1)
    _wait_scatter(0, 0)
    plsc.subcore_barrier()

    orow = c * N_PAD + rows0
    for k in range(ROWS_PT // C):
        pltpu.sync_copy(s_sh.at[pl.ds(rows0 + k * C, C)], gh_v0)
        pltpu.sync_copy(gh_v0, s_out.at[pl.ds(orow + k * C, C)])


def _make_sc_pass(t_bf16):
    t_shape = (C, D // 2) if t_bf16 else (C, D)
    t_dt = jnp.int32 if t_bf16 else jnp.float32
    return pl.kernel(
        functools.partial(_sc_pass_body, t_bf16),
        out_type=jax.ShapeDtypeStruct((NC * N_PAD, D), jnp.float32),
        mesh=_MESH,
        scratch_types=[
            pltpu.VMEM((4, C), jnp.int32),
            pltpu.VMEM((4, C), jnp.int32),
            pltpu.VMEM(t_shape, t_dt),
            pltpu.VMEM(t_shape, t_dt),
            pltpu.VMEM((C, D), jnp.float32),
            pltpu.VMEM((C, D), jnp.float32),
            pltpu.VMEM_SHARED((N_PAD, D), jnp.float32),
            pltpu.SemaphoreType.DMA,
            pltpu.SemaphoreType.DMA,
            pltpu.SemaphoreType.DMA,
            pltpu.SemaphoreType.DMA,
            pltpu.SemaphoreType.DMA,
            pltpu.SemaphoreType.DMA,
        ],
    )


_sc_pass = _make_sc_pass(False)
_sc_pass_b16 = _make_sc_pass(True)


def _sc_cnt_body(dst_hbm, cnt_out, dst_v, ones_v, cnt_sh):
    c = lax.axis_index("c")
    s = lax.axis_index("s")
    wid = c * NS + s
    rows0 = s * ROWS_PT
    pltpu.sync_copy(dst_hbm.at[wid], dst_v)

    def _cfill(val, i, _):
        for j in range(CW // VEC):
            ones_v[i, pl.ds(j * VEC, VEC)] = jnp.full((VEC,), val, jnp.float32)
        return 0
    lax.fori_loop(0, C, functools.partial(_cfill, 0.0), 0)
    for k in range(ROWS_PT // C):
        pltpu.sync_copy(ones_v, cnt_sh.at[pl.ds(rows0 + k * C, C)])
    lax.fori_loop(0, C, functools.partial(_cfill, 1.0), 0)
    plsc.subcore_barrier()

    def _chunk(i, _):
        pltpu.sync_copy(ones_v, cnt_sh.at[dst_v.at[i]], add=True)
        return 0
    lax.fori_loop(0, CHUNKS, _chunk, 0)
    plsc.subcore_barrier()

    orow = c * N_PAD + rows0
    for k in range(ROWS_PT // C):
        pltpu.sync_copy(cnt_sh.at[pl.ds(rows0 + k * C, C)], ones_v)
        pltpu.sync_copy(ones_v, cnt_out.at[pl.ds(orow + k * C, C)])


_sc_cnt = pl.kernel(
    _sc_cnt_body,
    out_type=jax.ShapeDtypeStruct((NC * N_PAD, CW), jnp.float32),
    mesh=_MESH,
    scratch_types=[
        pltpu.VMEM((CHUNKS, C), jnp.int32),
        pltpu.VMEM((C, CW), jnp.float32),
        pltpu.VMEM_SHARED((N_PAD, CW), jnp.float32),
    ],
)


BT = 2560


import numpy as _np
_PERM_A = _np.array([32 * (m // 16) + m % 16 for m in range(D // 2)])
_PERM_B = _PERM_A + VEC


def _bf16_bits(t):
    return lax.bitcast_convert_type(t.astype(jnp.bfloat16),
                                    jnp.uint16).astype(jnp.uint32)


def _et_body(e_ref, r1a, r1b, r2a, r2b, r3, c1a, c1b, c2a, c2b, c3,
             t1, t2, eo):
    e = e_ref[...]
    for (ra, rb, ca, cb, out) in ((r1a, r1b, c1a, c1b, t1),
                                  (r2a, r2b, c2a, c2b, t2)):
        a = jnp.dot(e, ra[...], preferred_element_type=jnp.float32) + ca[...]
        b = jnp.dot(e, rb[...], preferred_element_type=jnp.float32) + cb[...]
        w = (_bf16_bits(b) << 16) | _bf16_bits(a)
        out[...] = lax.bitcast_convert_type(w, jnp.int32)
    eo[...] = jnp.dot(e, r3[...], preferred_element_type=jnp.float32) + c3[...]


def _edge_transform(e0, R1, R2, R3, c1, c2, c3):
    hspec = pl.BlockSpec((D, D // 2), lambda i: (0, 0))
    wspec = pl.BlockSpec((D, D), lambda i: (0, 0))
    hbspec = pl.BlockSpec((1, D // 2), lambda i: (0, 0))
    bspec = pl.BlockSpec((1, D), lambda i: (0, 0))
    espec = pl.BlockSpec((BT, D), lambda i: (i, 0))
    pspec = pl.BlockSpec((BT, D // 2), lambda i: (i, 0))
    h = D // 2
    return pl.pallas_call(
        _et_body,
        grid=(E // BT,),
        in_specs=[espec, hspec, hspec, hspec, hspec, wspec,
                  hbspec, hbspec, hbspec, hbspec, bspec],
        out_specs=[pspec, pspec, espec],
        out_shape=[jax.ShapeDtypeStruct((E, D // 2), jnp.int32),
                   jax.ShapeDtypeStruct((E, D // 2), jnp.int32),
                   jax.ShapeDtypeStruct((E, D), jnp.float32)],
    )(e0, R1[:, _PERM_A], R1[:, _PERM_B], R2[:, _PERM_A], R2[:, _PERM_B], R3,
      c1[_PERM_A].reshape(1, h), c1[_PERM_B].reshape(1, h),
      c2[_PERM_A].reshape(1, h), c2[_PERM_B].reshape(1, h),
      c3.reshape(1, D))


def _nu_body(do_bn, s_ref, cnt_ref, x_ref, wi, bi, ws, bs, g_ref, b_ref, o_ref):
    sa = s_ref[...]
    S = sa[0:N] + sa[N_PAD:N_PAD + N]
    ca = cnt_ref[...]
    cnt = ca[0:N, 0:1] + ca[N_PAD:N_PAD + N, 0:1]
    meanh = S / jnp.maximum(cnt, 1.0)
    out = (jnp.dot(meanh, wi[...], preferred_element_type=jnp.float32)
           + jnp.minimum(cnt, 1.0) * bi[...]
           + jnp.dot(x_ref[...], ws[...], preferred_element_type=jnp.float32)
           + bs[...])
    if do_bn:
        mu = jnp.mean(out, axis=0, keepdims=True)
        var = jnp.mean((out - mu) ** 2, axis=0, keepdims=True)
        out = (out - mu) / jnp.sqrt(var + EPS) * g_ref[...] + b_ref[...]
        out = jnp.maximum(out, 0.0)
    o_ref[...] = out


def _node_update(Scat, cntcat, x, WinT, b_in, WselfT, b_self, bn_g, bn_b, do_bn):
    return pl.pallas_call(
        functools.partial(_nu_body, do_bn),
        out_shape=jax.ShapeDtypeStruct((N, D), jnp.float32),
    )(Scat, cntcat, x, WinT, b_in.reshape(1, D), WselfT,
      b_self.reshape(1, D), bn_g.reshape(1, D), bn_b.reshape(1, D))


def kernel(x, edge_index, edge_attr, params):
    p = params
    src = edge_index[0]
    dst = edge_index[1]

    W0T = p["W_rel_0"].T
    W1T = p["W_rel_1"].T
    W2T = p["W_rel_2"].T
    R1 = W0T
    c1 = p["b_rel_0"]
    R2 = R1 @ W1T
    c2 = c1 @ W1T + p["b_rel_1"]
    R3 = R2 @ W2T
    c3 = c2 @ W2T + p["b_rel_2"]

    t1, t2, e_out = _edge_transform(edge_attr, R1, R2, R3, c1, c2, c3)

    src3 = src.reshape(NW, CHUNKS, C)
    dst3 = dst.reshape(NW, CHUNKS, C)
    cnt = _sc_cnt(dst3)
    S0 = _sc_pass(edge_attr, src3, dst3, x)
    x1 = _node_update(S0, cnt, x, p["W_in_0"].T, p["b_in_0"],
                      p["W_self_0"].T, p["b_self_0"],
                      p["bn_g_0"], p["bn_b_0"], True)
    S1 = _sc_pass_b16(t1, src3, dst3, x1)
    x2 = _node_update(S1, cnt, x1, p["W_in_1"].T, p["b_in_1"],
                      p["W_self_1"].T, p["b_self_1"],
                      p["bn_g_1"], p["bn_b_1"], True)
    S2 = _sc_pass_b16(t2, src3, dst3, x2)
    x3 = _node_update(S2, cnt, x2, p["W_in_2"].T, p["b_in_2"],
                      p["W_self_2"].T, p["b_self_2"],
                      p["bn_g_1"], p["bn_b_1"], False)
    return (x3, e_out)


# speedup vs baseline: 5.5375x; 1.0080x over previous
import functools

import jax
import jax.numpy as jnp
from jax import lax
from jax.experimental import pallas as pl
from jax.experimental.pallas import tpu as pltpu
from jax.experimental.pallas import tpu_sc as plsc

N = 10000
E = 320000
D = 128
EPS = 1e-5

NC = 2
NS = 16
NW = NC * NS
PER_TILE = E // NW
C = 80
CHUNKS = PER_TILE // C
N_PAD = 10240
ROWS_PT = N_PAD // NS
CW = 128
VEC = 16


_MESH = plsc.VectorSubcoreMesh(core_axis_name="c", subcore_axis_name="s")


def _sc_pass_body(t_bf16, t_hbm, src_hbm, dst_hbm, x_hbm, s_out,
                  src_v, dst_v, t_v0, t_v1, gh_v0, gh_v1, s_sh,
                  sem_sd0, sem_sd1, sem_g0, sem_g1, sem_w0, sem_w1):
    c = lax.axis_index("c")
    s = lax.axis_index("s")
    wid = c * NS + s
    ebase = wid * PER_TILE
    rows0 = s * ROWS_PT
    t_v = (t_v0, t_v1)
    gh_v = (gh_v0, gh_v1)
    sem_sd = (sem_sd0, sem_sd1)
    sem_g = (sem_g0, sem_g1)
    sem_w = (sem_w0, sem_w1)

    def _t_slice(i):
        return t_hbm.at[pl.ds(ebase + i * C, C)]

    def _issue_loads(i, ph, b):
        pltpu.async_copy(src_hbm.at[wid, i], src_v.at[ph], sem_sd[b])
        pltpu.async_copy(dst_hbm.at[wid, i], dst_v.at[ph], sem_sd[b])
        pltpu.async_copy(_t_slice(i), t_v[b], sem_sd[b])

    def _wait_loads(i, ph, b):
        pltpu.make_async_copy(src_hbm.at[wid, i], src_v.at[ph], sem_sd[b]).wait()
        pltpu.make_async_copy(dst_hbm.at[wid, i], dst_v.at[ph], sem_sd[b]).wait()
        pltpu.make_async_copy(_t_slice(i), t_v[b], sem_sd[b]).wait()

    def _issue_gather(ph, b):
        pltpu.async_copy(x_hbm.at[src_v.at[ph]], gh_v[b], sem_g[b])

    def _wait_gather(ph, b):
        pltpu.make_async_copy(x_hbm.at[src_v.at[ph]], gh_v[b], sem_g[b]).wait()

    def _issue_scatter(ph, b):
        pltpu.async_copy(gh_v[b], s_sh.at[dst_v.at[ph]], sem_w[b], add=True)

    def _wait_scatter(ph, b):
        pltpu.make_async_copy(gh_v[b], s_sh.at[dst_v.at[ph]], sem_w[b]).wait()

    def _mul(b):
        tb, hb = t_v[b], gh_v[b]

        def _mulrow(r, _):
            r4 = r * 4
            for rr in range(4):
                if t_bf16:
                    vsh = jnp.full((VEC,), 16, jnp.int32)
                    vmask = jnp.full((VEC,), -65536, jnp.int32)
                    for j in range(D // (2 * VEC)):
                        w = tb[r4 + rr, pl.ds(j * VEC, VEC)]
                        ta = lax.bitcast_convert_type(
                            lax.shift_left(w, vsh), jnp.float32)
                        tbb = lax.bitcast_convert_type(
                            lax.bitwise_and(w, vmask), jnp.float32)
                        sa = pl.ds(j * 2 * VEC, VEC)
                        sb = pl.ds(j * 2 * VEC + VEC, VEC)
                        hb[r4 + rr, sa] = ta * hb[r4 + rr, sa]
                        hb[r4 + rr, sb] = tbb * hb[r4 + rr, sb]
                else:
                    for j in range(D // VEC):
                        sl = pl.ds(j * VEC, VEC)
                        hb[r4 + rr, sl] = tb[r4 + rr, sl] * hb[r4 + rr, sl]
            return 0
        lax.fori_loop(0, C // 4, _mulrow, 0)

    def _zfill(i, _):
        for j in range(D // VEC):
            gh_v0[i, pl.ds(j * VEC, VEC)] = jnp.zeros((VEC,), jnp.float32)
        return 0
    lax.fori_loop(0, C, _zfill, 0)
    for k in range(ROWS_PT // C):
        pltpu.async_copy(gh_v0, s_sh.at[pl.ds(rows0 + k * C, C)], sem_w0)
    for k in range(ROWS_PT // C):
        pltpu.make_async_copy(gh_v0, s_sh.at[pl.ds(rows0 + k * C, C)],
                              sem_w0).wait()
    plsc.subcore_barrier()

    _issue_loads(0, 0, 0)
    _issue_loads(1, 1, 1)
    _wait_loads(0, 0, 0)
    _issue_gather(0, 0)

    def _quad(qi, _):
        for u in range(4):
            b = u % 2
            i = qi * 4 + u
            _wait_gather(u, b)
            _mul(b)
            _issue_scatter(u, b)

            @pl.when(i + 2 < CHUNKS)
            def _():
                _issue_loads(i + 2, (u + 2) % 4, b)
            _wait_loads(i + 1, (u + 1) % 4, 1 - b)
            if u == 0:
                @pl.when(qi >= 1)
                def _():
                    _wait_scatter((u + 3) % 4, 1 - b)
            else:
                _wait_scatter((u + 3) % 4, 1 - b)
            _issue_gather((u + 1) % 4, 1 - b)
        return 0
    lax.fori_loop(0, CHUNKS // 4, _quad, 0)

    _wait_gather(0, 0)
    _mul(0)
    _issue_scatter(0, 0)
    _wait_scatter(3, 1)
    _wait_scatter(0, 0)
    plsc.subcore_barrier()

    orow = c * N_PAD + rows0
    K = ROWS_PT // C

    def _dread(k, b):
        return pltpu.make_async_copy(
            s_sh.at[pl.ds(rows0 + k * C, C)], gh_v[b], sem_g[b])

    def _dwrit(k, b):
        return pltpu.make_async_copy(
            gh_v[b], s_out.at[pl.ds(orow + k * C, C)], sem_w[b])

    _dread(0, 0).start()
    for k in range(K):
        b = k % 2
        _dread(k, b).wait()
        _dwrit(k, b).start()
        if k + 1 < K:
            if k >= 1:
                _dwrit(k - 1, 1 - b).wait()
            _dread(k + 1, 1 - b).start()
    _dwrit(K - 2, K % 2).wait()
    _dwrit(K - 1, 1 - K % 2).wait()


def _make_sc_pass(t_bf16):
    t_shape = (C, D // 2) if t_bf16 else (C, D)
    t_dt = jnp.int32 if t_bf16 else jnp.float32
    return pl.kernel(
        functools.partial(_sc_pass_body, t_bf16),
        out_type=jax.ShapeDtypeStruct((NC * N_PAD, D), jnp.float32),
        mesh=_MESH,
        scratch_types=[
            pltpu.VMEM((4, C), jnp.int32),
            pltpu.VMEM((4, C), jnp.int32),
            pltpu.VMEM(t_shape, t_dt),
            pltpu.VMEM(t_shape, t_dt),
            pltpu.VMEM((C, D), jnp.float32),
            pltpu.VMEM((C, D), jnp.float32),
            pltpu.VMEM_SHARED((N_PAD, D), jnp.float32),
            pltpu.SemaphoreType.DMA,
            pltpu.SemaphoreType.DMA,
            pltpu.SemaphoreType.DMA,
            pltpu.SemaphoreType.DMA,
            pltpu.SemaphoreType.DMA,
            pltpu.SemaphoreType.DMA,
        ],
    )


_sc_pass = _make_sc_pass(False)
_sc_pass_b16 = _make_sc_pass(True)


def _sc_cnt_body(dst_hbm, cnt_out, dst_v, ones_v, cnt_sh, sem):
    c = lax.axis_index("c")
    s = lax.axis_index("s")
    wid = c * NS + s
    rows0 = s * ROWS_PT
    pltpu.sync_copy(dst_hbm.at[wid], dst_v)

    def _cfill(val, i, _):
        for j in range(CW // VEC):
            ones_v[i, pl.ds(j * VEC, VEC)] = jnp.full((VEC,), val, jnp.float32)
        return 0
    lax.fori_loop(0, C, functools.partial(_cfill, 0.0), 0)
    for k in range(ROWS_PT // C):
        pltpu.sync_copy(ones_v, cnt_sh.at[pl.ds(rows0 + k * C, C)])
    lax.fori_loop(0, C, functools.partial(_cfill, 1.0), 0)
    plsc.subcore_barrier()

    pltpu.async_copy(ones_v, cnt_sh.at[dst_v.at[0]], sem, add=True)

    def _chunk(i, _):
        @pl.when(i + 1 < CHUNKS)
        def _():
            pltpu.async_copy(ones_v, cnt_sh.at[dst_v.at[i + 1]], sem, add=True)
        pltpu.make_async_copy(ones_v, cnt_sh.at[dst_v.at[i]], sem).wait()
        return 0
    lax.fori_loop(0, CHUNKS, _chunk, 0)
    plsc.subcore_barrier()

    orow = c * N_PAD + rows0
    for k in range(ROWS_PT // C):
        pltpu.sync_copy(cnt_sh.at[pl.ds(rows0 + k * C, C)], ones_v)
        pltpu.sync_copy(ones_v, cnt_out.at[pl.ds(orow + k * C, C)])


_sc_cnt = pl.kernel(
    _sc_cnt_body,
    out_type=jax.ShapeDtypeStruct((NC * N_PAD, CW), jnp.float32),
    mesh=_MESH,
    scratch_types=[
        pltpu.VMEM((CHUNKS, C), jnp.int32),
        pltpu.VMEM((C, CW), jnp.float32),
        pltpu.VMEM_SHARED((N_PAD, CW), jnp.float32),
        pltpu.SemaphoreType.DMA,
    ],
)


BT = 2560


import numpy as _np
_PERM_A = _np.array([32 * (m // 16) + m % 16 for m in range(D // 2)])
_PERM_B = _PERM_A + VEC


def _bf16_bits(t):
    return lax.bitcast_convert_type(t.astype(jnp.bfloat16),
                                    jnp.uint16).astype(jnp.uint32)


def _et_body(e_ref, r1a, r1b, r2a, r2b, r3, c1a, c1b, c2a, c2b, c3,
             t1, t2, eo):
    e = e_ref[...]
    for (ra, rb, ca, cb, out) in ((r1a, r1b, c1a, c1b, t1),
                                  (r2a, r2b, c2a, c2b, t2)):
        a = jnp.dot(e, ra[...], preferred_element_type=jnp.float32) + ca[...]
        b = jnp.dot(e, rb[...], preferred_element_type=jnp.float32) + cb[...]
        w = (_bf16_bits(b) << 16) | _bf16_bits(a)
        out[...] = lax.bitcast_convert_type(w, jnp.int32)
    eo[...] = jnp.dot(e, r3[...], preferred_element_type=jnp.float32) + c3[...]


def _edge_transform(e0, R1, R2, R3, c1, c2, c3):
    hspec = pl.BlockSpec((D, D // 2), lambda i: (0, 0))
    wspec = pl.BlockSpec((D, D), lambda i: (0, 0))
    hbspec = pl.BlockSpec((1, D // 2), lambda i: (0, 0))
    bspec = pl.BlockSpec((1, D), lambda i: (0, 0))
    espec = pl.BlockSpec((BT, D), lambda i: (i, 0))
    pspec = pl.BlockSpec((BT, D // 2), lambda i: (i, 0))
    h = D // 2
    return pl.pallas_call(
        _et_body,
        grid=(E // BT,),
        in_specs=[espec, hspec, hspec, hspec, hspec, wspec,
                  hbspec, hbspec, hbspec, hbspec, bspec],
        out_specs=[pspec, pspec, espec],
        out_shape=[jax.ShapeDtypeStruct((E, D // 2), jnp.int32),
                   jax.ShapeDtypeStruct((E, D // 2), jnp.int32),
                   jax.ShapeDtypeStruct((E, D), jnp.float32)],
    )(e0, R1[:, _PERM_A], R1[:, _PERM_B], R2[:, _PERM_A], R2[:, _PERM_B], R3,
      c1[_PERM_A].reshape(1, h), c1[_PERM_B].reshape(1, h),
      c2[_PERM_A].reshape(1, h), c2[_PERM_B].reshape(1, h),
      c3.reshape(1, D))


def _nu_body(do_bn, s_ref, cnt_ref, x_ref, wi, bi, ws, bs, g_ref, b_ref, o_ref):
    sa = s_ref[...]
    S = sa[0:N] + sa[N_PAD:N_PAD + N]
    ca = cnt_ref[...]
    cnt = ca[0:N, 0:1] + ca[N_PAD:N_PAD + N, 0:1]
    meanh = S / jnp.maximum(cnt, 1.0)
    out = (jnp.dot(meanh, wi[...], preferred_element_type=jnp.float32)
           + jnp.minimum(cnt, 1.0) * bi[...]
           + jnp.dot(x_ref[...], ws[...], preferred_element_type=jnp.float32)
           + bs[...])
    if do_bn:
        mu = jnp.mean(out, axis=0, keepdims=True)
        var = jnp.mean((out - mu) ** 2, axis=0, keepdims=True)
        out = (out - mu) / jnp.sqrt(var + EPS) * g_ref[...] + b_ref[...]
        out = jnp.maximum(out, 0.0)
    o_ref[...] = out


def _node_update(Scat, cntcat, x, WinT, b_in, WselfT, b_self, bn_g, bn_b, do_bn):
    return pl.pallas_call(
        functools.partial(_nu_body, do_bn),
        out_shape=jax.ShapeDtypeStruct((N, D), jnp.float32),
    )(Scat, cntcat, x, WinT, b_in.reshape(1, D), WselfT,
      b_self.reshape(1, D), bn_g.reshape(1, D), bn_b.reshape(1, D))


def kernel(x, edge_index, edge_attr, params):
    p = params
    src = edge_index[0]
    dst = edge_index[1]

    W0T = p["W_rel_0"].T
    W1T = p["W_rel_1"].T
    W2T = p["W_rel_2"].T
    R1 = W0T
    c1 = p["b_rel_0"]
    R2 = R1 @ W1T
    c2 = c1 @ W1T + p["b_rel_1"]
    R3 = R2 @ W2T
    c3 = c2 @ W2T + p["b_rel_2"]

    t1, t2, e_out = _edge_transform(edge_attr, R1, R2, R3, c1, c2, c3)

    src3 = src.reshape(NW, CHUNKS, C)
    dst3 = dst.reshape(NW, CHUNKS, C)
    cnt = _sc_cnt(dst3)
    S0 = _sc_pass(edge_attr, src3, dst3, x)
    x1 = _node_update(S0, cnt, x, p["W_in_0"].T, p["b_in_0"],
                      p["W_self_0"].T, p["b_self_0"],
                      p["bn_g_0"], p["bn_b_0"], True)
    S1 = _sc_pass_b16(t1, src3, dst3, x1)
    x2 = _node_update(S1, cnt, x1, p["W_in_1"].T, p["b_in_1"],
                      p["W_self_1"].T, p["b_self_1"],
                      p["bn_g_1"], p["bn_b_1"], True)
    S2 = _sc_pass_b16(t2, src3, dst3, x2)
    x3 = _node_update(S2, cnt, x2, p["W_in_2"].T, p["b_in_2"],
                      p["W_self_2"].T, p["b_self_2"],
                      p["bn_g_1"], p["bn_b_1"], False)
    return (x3, e_out)
